# Initial kernel scaffold; baseline (speedup 1.0000x reference)
import jax
import jax.numpy as jnp
from jax.experimental import pallas as pl


def kernel(q, keys, values):
    raise NotImplementedError("write your pallas kernel here")


# speedup vs baseline: 2.5301x; 2.5301x over previous
import jax
import jax.numpy as jnp
from jax import lax
from jax.experimental import pallas as pl
from jax.experimental.pallas import tpu as pltpu

CAP = 100000
D = 128
Q = 4096
K = 5
CHUNK = 8
NCAND = K * CHUNK
KPAD = 102400
BK = 2048
NKB = KPAD // BK
CPB = BK // CHUNK
NCHUNK = CAP // CHUNK
NCHUNK_PAD = KPAD // CHUNK
QL = 512
NQB = Q // QL
GQ = 8

_NEG = float("-inf")
_IMAX = 2**31 - 1


def _phaseAB_body(kn_ref, q_ref, out_ref, cmax_ref):
    kb = pl.program_id(1)
    u = lax.dot_general(kn_ref[...], q_ref[...], (((1,), (1,)), ((), ())),
                        preferred_element_type=jnp.float32)
    m = jnp.max(u.reshape(CPB, CHUNK, QL), axis=1)
    gchunk = kb * CPB + lax.broadcasted_iota(jnp.int32, (CPB, QL), 0)
    m = jnp.where(gchunk < NCHUNK, m, _NEG)
    cmax_ref[pl.ds(kb * CPB, CPB), :] = m

    @pl.when(kb == NKB - 1)
    def _phaseB():
        rows = lax.broadcasted_iota(jnp.int32, (NCHUNK_PAD, QL), 0)
        for t in range(K):
            c = cmax_ref[...]
            bidx = jnp.argmax(c, axis=0).astype(jnp.int32)
            out_ref[0, t, :] = bidx
            if t < K - 1:
                cmax_ref[...] = jnp.where(rows == bidx[None, :], _NEG, c)


def _phaseAB(qn, kn):
    return pl.pallas_call(
        _phaseAB_body,
        grid=(NQB, NKB),
        in_specs=[
            pl.BlockSpec((BK, D), lambda ql, kb: (kb, 0)),
            pl.BlockSpec((QL, D), lambda ql, kb: (ql, 0)),
        ],
        out_specs=pl.BlockSpec((1, K, QL), lambda ql, kb: (ql, 0, 0)),
        out_shape=jax.ShapeDtypeStruct((NQB, K, QL), jnp.int32),
        scratch_shapes=[pltpu.VMEM((NCHUNK_PAD, QL), jnp.float32)],
        compiler_params=pltpu.CompilerParams(
            dimension_semantics=("arbitrary", "arbitrary"),
            vmem_limit_bytes=128 * 1024 * 1024,
        ),
    )(kn, qn)


def _pb2_body(qe_ref, kc_ref, cand_ref, sel_ref):
    n = GQ * NCAND
    p = lax.dot_general(qe_ref[...], kc_ref[...], (((1,), (1,)), ((), ())),
                        preferred_element_type=jnp.float32)
    ir = lax.broadcasted_iota(jnp.int32, (n, n), 0)
    ic = lax.broadcasted_iota(jnp.int32, (n, n), 1)
    s = jnp.sum(jnp.where(ir == ic, p, 0.0), axis=1).reshape(GQ, NCAND)
    cand = cand_ref[...]
    for t in range(K):
        mx = jnp.max(s, axis=1)
        kmask = jnp.where(s == mx[:, None], cand, _IMAX)
        pick = jnp.min(kmask, axis=1)
        sel_ref[:, t] = pick
        if t < K - 1:
            s = jnp.where(kmask == pick[:, None], _NEG, s)


def _pb2(qn_exp, kc, cand):
    return pl.pallas_call(
        _pb2_body,
        grid=(Q // GQ,),
        in_specs=[
            pl.BlockSpec((GQ * NCAND, D), lambda g: (g, 0)),
            pl.BlockSpec((GQ * NCAND, D), lambda g: (g, 0)),
            pl.BlockSpec((GQ, NCAND), lambda g: (g, 0)),
        ],
        out_specs=pl.BlockSpec((GQ, K), lambda g: (g, 0)),
        out_shape=jax.ShapeDtypeStruct((Q, K), jnp.int32),
    )(qn_exp, kc, cand)


def kernel(q, keys, values):
    eps = 1e-8
    qn = q / jnp.maximum(jnp.linalg.norm(q, axis=-1, keepdims=True), eps)
    kn = keys / jnp.maximum(jnp.linalg.norm(keys, axis=-1, keepdims=True), eps)
    kn_p = jnp.pad(kn, ((0, KPAD - CAP), (0, 0)))

    t5 = _phaseAB(qn, kn_p)
    top5c = t5.transpose(0, 2, 1).reshape(Q, K)

    cand = (top5c[:, :, None] * CHUNK
            + jnp.arange(CHUNK, dtype=jnp.int32)[None, None, :]).reshape(Q, NCAND)
    kc = kn_p[cand.reshape(-1)]
    qn_exp = jnp.repeat(qn, NCAND, axis=0)

    sel = _pb2(qn_exp, kc, cand)

    vrows = values[sel.reshape(-1)].reshape(Q, K, D)
    return vrows.mean(axis=1)


# speedup vs baseline: 2.5796x; 1.0195x over previous
import functools

import jax
import jax.numpy as jnp
from jax import lax
from jax.experimental import pallas as pl
from jax.experimental.pallas import tpu as pltpu
from jax.experimental.pallas import tpu_sc as plsc

CAP = 100000
D = 128
Q = 4096
K = 5
CHUNK = 8
NCAND = K * CHUNK
KPAD = 102400
BK = 2048
NKB = KPAD // BK
CPB = BK // CHUNK
NCHUNK = CAP // CHUNK
NCHUNK_PAD = KPAD // CHUNK
QL = 512
NQB = Q // QL
GQ = 8

_NEG = float("-inf")
_IMAX = 2**31 - 1


def _phaseAB_body(kn_ref, q_ref, out_ref, cmax_ref):
    kb = pl.program_id(1)
    u = lax.dot_general(kn_ref[...], q_ref[...], (((1,), (1,)), ((), ())),
                        preferred_element_type=jnp.float32)
    m = jnp.max(u.reshape(CPB, CHUNK, QL), axis=1)
    gchunk = kb * CPB + lax.broadcasted_iota(jnp.int32, (CPB, QL), 0)
    m = jnp.where(gchunk < NCHUNK, m, _NEG)
    cmax_ref[pl.ds(kb * CPB, CPB), :] = m

    @pl.when(kb == NKB - 1)
    def _phaseB():
        rows = lax.broadcasted_iota(jnp.int32, (NCHUNK_PAD, QL), 0)
        for t in range(K):
            c = cmax_ref[...]
            bidx = jnp.argmax(c, axis=0).astype(jnp.int32)
            for j in range(CHUNK):
                out_ref[0, t * CHUNK + j, :] = bidx * CHUNK + j
            if t < K - 1:
                cmax_ref[...] = jnp.where(rows == bidx[None, :], _NEG, c)


def _phaseAB(qn, kn):
    return pl.pallas_call(
        _phaseAB_body,
        grid=(NQB, NKB),
        in_specs=[
            pl.BlockSpec((BK, D), lambda ql, kb: (kb, 0)),
            pl.BlockSpec((QL, D), lambda ql, kb: (ql, 0)),
        ],
        out_specs=pl.BlockSpec((1, NCAND, QL), lambda ql, kb: (ql, 0, 0)),
        out_shape=jax.ShapeDtypeStruct((NQB, NCAND, QL), jnp.int32),
        scratch_shapes=[pltpu.VMEM((NCHUNK_PAD, QL), jnp.float32)],
        compiler_params=pltpu.CompilerParams(
            dimension_semantics=("arbitrary", "arbitrary"),
            vmem_limit_bytes=128 * 1024 * 1024,
        ),
    )(kn, qn)


def _pb2_body(qe_ref, kc_ref, cand_ref, sel_ref):
    n = GQ * NCAND
    p = lax.dot_general(qe_ref[...], kc_ref[...], (((1,), (1,)), ((), ())),
                        preferred_element_type=jnp.float32)
    ir = lax.broadcasted_iota(jnp.int32, (n, n), 0)
    ic = lax.broadcasted_iota(jnp.int32, (n, n), 1)
    s = jnp.sum(jnp.where(ir == ic, p, 0.0), axis=1).reshape(GQ, NCAND)
    cand = cand_ref[...]
    for t in range(K):
        mx = jnp.max(s, axis=1)
        kmask = jnp.where(s == mx[:, None], cand, _IMAX)
        pick = jnp.min(kmask, axis=1)
        sel_ref[:, t] = pick
        if t < K - 1:
            s = jnp.where(kmask == pick[:, None], _NEG, s)


def _pb2(qn_exp, kc, cand):
    return pl.pallas_call(
        _pb2_body,
        grid=(Q // GQ,),
        in_specs=[
            pl.BlockSpec((GQ * NCAND, D), lambda g: (g, 0)),
            pl.BlockSpec((GQ * NCAND, D), lambda g: (g, 0)),
            pl.BlockSpec((GQ, NCAND), lambda g: (g, 0)),
        ],
        out_specs=pl.BlockSpec((GQ, K), lambda g: (g, 0)),
        out_shape=jax.ShapeDtypeStruct((Q, K), jnp.int32),
    )(qn_exp, kc, cand)


_NW = 32
_QPW = Q // _NW
_SB1 = 16


def _sc_gather_cand(cand_flat, kn_p):
    mesh = plsc.VectorSubcoreMesh(core_axis_name="c", subcore_axis_name="s")

    @functools.partial(
        pl.kernel, mesh=mesh,
        out_type=jax.ShapeDtypeStruct((Q * NCAND, D), jnp.float32),
        scratch_types=[
            pltpu.VMEM((_SB1 * NCAND,), jnp.int32),
            pltpu.VMEM((_SB1 * NCAND, D), jnp.float32),
            pltpu.SemaphoreType.DMA,
        ],
    )
    def sc1(cand_hbm, kn_hbm, kc_hbm, idx_v, rows_v, sem):
        wid = lax.axis_index("s") * 2 + lax.axis_index("c")
        for b in range(_QPW // _SB1):
            base = (wid * _QPW + b * _SB1) * NCAND
            pltpu.sync_copy(cand_hbm.at[pl.ds(base, _SB1 * NCAND)], idx_v)
            pltpu.async_copy(kn_hbm.at[idx_v], rows_v, sem).wait()
            pltpu.sync_copy(rows_v, kc_hbm.at[pl.ds(base, _SB1 * NCAND)])

    return sc1(cand_flat, kn_p)


def _sc_gather_values(sel_flat, values):
    mesh = plsc.VectorSubcoreMesh(core_axis_name="c", subcore_axis_name="s")
    rpw = _QPW * K

    @functools.partial(
        pl.kernel, mesh=mesh,
        out_type=jax.ShapeDtypeStruct((Q * K, D), jnp.float32),
        scratch_types=[
            pltpu.VMEM((rpw,), jnp.int32),
            pltpu.VMEM((rpw, D), jnp.float32),
            pltpu.SemaphoreType.DMA,
        ],
    )
    def sc2(sel_hbm, val_hbm, out_hbm, sidx_v, vr_v, sem):
        wid = lax.axis_index("s") * 2 + lax.axis_index("c")
        base = wid * rpw
        pltpu.sync_copy(sel_hbm.at[pl.ds(base, rpw)], sidx_v)
        pltpu.async_copy(val_hbm.at[sidx_v], vr_v, sem).wait()
        pltpu.sync_copy(vr_v, out_hbm.at[pl.ds(base, rpw)])

    return sc2(sel_flat, values)


def _mean_body(v_ref, out_ref):
    x = v_ref[...]
    s = x[:, 0, :] + x[:, 1, :] + x[:, 2, :] + x[:, 3, :] + x[:, 4, :]
    out_ref[...] = s * (1.0 / K)


def _mean5(vrows):
    gq = 256
    return pl.pallas_call(
        _mean_body,
        grid=(Q // gq,),
        in_specs=[pl.BlockSpec((gq, K, D), lambda g: (g, 0, 0))],
        out_specs=pl.BlockSpec((gq, D), lambda g: (g, 0)),
        out_shape=jax.ShapeDtypeStruct((Q, D), jnp.float32),
    )(vrows)


def kernel(q, keys, values):
    eps = 1e-8
    qn = q / jnp.maximum(jnp.linalg.norm(q, axis=-1, keepdims=True), eps)
    kn = keys / jnp.maximum(jnp.linalg.norm(keys, axis=-1, keepdims=True), eps)
    kn_p = jnp.pad(kn, ((0, KPAD - CAP), (0, 0)))

    t5 = _phaseAB(qn, kn_p)
    cand = t5.transpose(0, 2, 1).reshape(Q, NCAND)

    kc = _sc_gather_cand(cand.reshape(-1), kn_p)
    qn_exp = jnp.repeat(qn, NCAND, axis=0)

    sel = _pb2(qn_exp, kc, cand)

    vrows = _sc_gather_values(sel.reshape(-1), values)
    return _mean5(vrows.reshape(Q, K, D))


# speedup vs baseline: 5.6829x; 2.2030x over previous
import functools

import jax
import jax.numpy as jnp
from jax import lax
from jax.experimental import pallas as pl
from jax.experimental.pallas import tpu as pltpu
from jax.experimental.pallas import tpu_sc as plsc

CAP = 100000
D = 128
Q = 4096
K = 5
CHUNK = 8
NCAND = K * CHUNK
KPAD = 102400
BK = 2048
NKB = KPAD // BK
CPB = BK // CHUNK
NCHUNK = CAP // CHUNK
NCHUNK_PAD = KPAD // CHUNK
QL = 512
NQB = Q // QL
GQ = 64

_NEG = float("-inf")
_IMAX = 2**31 - 1


def _phaseAB_body(kn_ref, q_ref, out_ref, cmax_ref):
    kb = pl.program_id(1)
    u = lax.dot_general(kn_ref[...], q_ref[...], (((1,), (1,)), ((), ())),
                        preferred_element_type=jnp.float32)
    m = jnp.max(u.reshape(CPB, CHUNK, QL), axis=1)
    gchunk = kb * CPB + lax.broadcasted_iota(jnp.int32, (CPB, QL), 0)
    m = jnp.where(gchunk < NCHUNK, m, _NEG)
    cmax_ref[pl.ds(kb * CPB, CPB), :] = m

    @pl.when(kb == NKB - 1)
    def _phaseB():
        rows = lax.broadcasted_iota(jnp.int32, (NCHUNK_PAD, QL), 0)
        for t in range(K):
            c = cmax_ref[...]
            bidx = jnp.argmax(c, axis=0).astype(jnp.int32)
            for j in range(CHUNK):
                out_ref[0, t * CHUNK + j, :] = bidx * CHUNK + j
            if t < K - 1:
                cmax_ref[...] = jnp.where(rows == bidx[None, :], _NEG, c)


def _phaseAB(qn, kn):
    return pl.pallas_call(
        _phaseAB_body,
        grid=(NQB, NKB),
        in_specs=[
            pl.BlockSpec((BK, D), lambda ql, kb: (kb, 0)),
            pl.BlockSpec((QL, D), lambda ql, kb: (ql, 0)),
        ],
        out_specs=pl.BlockSpec((1, NCAND, QL), lambda ql, kb: (ql, 0, 0)),
        out_shape=jax.ShapeDtypeStruct((NQB, NCAND, QL), jnp.int32),
        scratch_shapes=[pltpu.VMEM((NCHUNK_PAD, QL), jnp.float32)],
        compiler_params=pltpu.CompilerParams(
            dimension_semantics=("arbitrary", "arbitrary"),
            vmem_limit_bytes=128 * 1024 * 1024,
        ),
    )(kn, qn)


def _pb2_body(q_ref, kc_ref, cand_ref, sel_ref):
    p = lax.dot_general(q_ref[...], kc_ref[...], (((1,), (1,)), ((), ())),
                        preferred_element_type=jnp.float32)
    qrow = lax.broadcasted_iota(jnp.int32, (GQ, NCAND), 0)
    s = jnp.zeros((GQ, NCAND), jnp.float32)
    for a in range(GQ):
        stripe = p[:, a * NCAND:(a + 1) * NCAND]
        s = s + jnp.where(qrow == a, stripe, 0.0)
    cand = cand_ref[...]
    for t in range(K):
        mx = jnp.max(s, axis=1)
        kmask = jnp.where(s == mx[:, None], cand, _IMAX)
        pick = jnp.min(kmask, axis=1)
        sel_ref[:, t] = pick
        if t < K - 1:
            s = jnp.where(kmask == pick[:, None], _NEG, s)


def _pb2(qn, kc, cand):
    return pl.pallas_call(
        _pb2_body,
        grid=(Q // GQ,),
        in_specs=[
            pl.BlockSpec((GQ, D), lambda g: (g, 0)),
            pl.BlockSpec((GQ * NCAND, D), lambda g: (g, 0)),
            pl.BlockSpec((GQ, NCAND), lambda g: (g, 0)),
        ],
        out_specs=pl.BlockSpec((GQ, K), lambda g: (g, 0)),
        out_shape=jax.ShapeDtypeStruct((Q, K), jnp.int32),
    )(qn, kc, cand)


_NW = 32
_QPW = Q // _NW
_SB1 = 16


def _sc_gather_cand(cand_flat, kn_p):
    mesh = plsc.VectorSubcoreMesh(core_axis_name="c", subcore_axis_name="s")

    @functools.partial(
        pl.kernel, mesh=mesh,
        out_type=jax.ShapeDtypeStruct((Q * NCAND, D), jnp.float32),
        scratch_types=[
            pltpu.VMEM((_SB1 * NCAND,), jnp.int32),
            pltpu.VMEM((_SB1 * NCAND, D), jnp.float32),
            pltpu.SemaphoreType.DMA,
        ],
    )
    def sc1(cand_hbm, kn_hbm, kc_hbm, idx_v, rows_v, sem):
        wid = lax.axis_index("s") * 2 + lax.axis_index("c")
        for b in range(_QPW // _SB1):
            base = (wid * _QPW + b * _SB1) * NCAND
            pltpu.sync_copy(cand_hbm.at[pl.ds(base, _SB1 * NCAND)], idx_v)
            pltpu.async_copy(kn_hbm.at[idx_v], rows_v, sem).wait()
            pltpu.sync_copy(rows_v, kc_hbm.at[pl.ds(base, _SB1 * NCAND)])

    return sc1(cand_flat, kn_p)


def _sc_gather_values(sel_flat, values):
    mesh = plsc.VectorSubcoreMesh(core_axis_name="c", subcore_axis_name="s")
    rpw = _QPW * K

    @functools.partial(
        pl.kernel, mesh=mesh,
        out_type=jax.ShapeDtypeStruct((Q * K, D), jnp.float32),
        scratch_types=[
            pltpu.VMEM((rpw,), jnp.int32),
            pltpu.VMEM((rpw, D), jnp.float32),
            pltpu.SemaphoreType.DMA,
        ],
    )
    def sc2(sel_hbm, val_hbm, out_hbm, sidx_v, vr_v, sem):
        wid = lax.axis_index("s") * 2 + lax.axis_index("c")
        base = wid * rpw
        pltpu.sync_copy(sel_hbm.at[pl.ds(base, rpw)], sidx_v)
        pltpu.async_copy(val_hbm.at[sidx_v], vr_v, sem).wait()
        pltpu.sync_copy(vr_v, out_hbm.at[pl.ds(base, rpw)])

    return sc2(sel_flat, values)


def _mean_body(v_ref, out_ref):
    x = v_ref[...]
    s = x[:, 0, :] + x[:, 1, :] + x[:, 2, :] + x[:, 3, :] + x[:, 4, :]
    out_ref[...] = s * (1.0 / K)


def _mean5(vrows):
    gq = 256
    return pl.pallas_call(
        _mean_body,
        grid=(Q // gq,),
        in_specs=[pl.BlockSpec((gq, K, D), lambda g: (g, 0, 0))],
        out_specs=pl.BlockSpec((gq, D), lambda g: (g, 0)),
        out_shape=jax.ShapeDtypeStruct((Q, D), jnp.float32),
    )(vrows)


def kernel(q, keys, values):
    eps = 1e-8
    qn = q / jnp.maximum(jnp.linalg.norm(q, axis=-1, keepdims=True), eps)
    kn = keys / jnp.maximum(jnp.linalg.norm(keys, axis=-1, keepdims=True), eps)
    kn_p = jnp.pad(kn, ((0, KPAD - CAP), (0, 0)))

    t5 = _phaseAB(qn, kn_p)
    cand = t5.transpose(0, 2, 1).reshape(Q, NCAND)

    kc = _sc_gather_cand(cand.reshape(-1), kn_p)
    sel = _pb2(qn, kc, cand)

    vrows = _sc_gather_values(sel.reshape(-1), values)
    return _mean5(vrows.reshape(Q, K, D))


# speedup vs baseline: 5.8299x; 1.0259x over previous
import functools

import jax
import jax.numpy as jnp
from jax import lax
from jax.experimental import pallas as pl
from jax.experimental.pallas import tpu as pltpu
from jax.experimental.pallas import tpu_sc as plsc

CAP = 100000
D = 128
Q = 4096
K = 5
CHUNK = 8
NCAND = K * CHUNK
KPAD = 102400
BK = 2048
NKB = KPAD // BK
CPB = BK // CHUNK
NCHUNK = CAP // CHUNK
NCHUNK_PAD = KPAD // CHUNK
QL = 512
NQB = Q // QL
GQ = 64

_NEG = float("-inf")
_IMAX = 2**31 - 1


def _phaseAB_body(kn_ref, q_ref, out_ref, cmax_ref):
    kb = pl.program_id(1)
    u = lax.dot_general(kn_ref[...], q_ref[...], (((1,), (1,)), ((), ())),
                        preferred_element_type=jnp.float32)
    m = jnp.max(u.reshape(CPB, CHUNK, QL), axis=1)
    gchunk = kb * CPB + lax.broadcasted_iota(jnp.int32, (CPB, QL), 0)
    m = jnp.where(gchunk < NCHUNK, m, _NEG)
    cmax_ref[pl.ds(kb * CPB, CPB), :] = m

    @pl.when(kb == NKB - 1)
    def _phaseB():
        rows = lax.broadcasted_iota(jnp.int32, (NCHUNK_PAD, QL), 0)
        for t in range(K):
            c = cmax_ref[...]
            bidx = jnp.argmax(c, axis=0).astype(jnp.int32)
            for j in range(CHUNK):
                out_ref[0, t * CHUNK + j, :] = bidx * CHUNK + j
            if t < K - 1:
                cmax_ref[...] = jnp.where(rows == bidx[None, :], _NEG, c)


def _phaseAB(qn, kn):
    return pl.pallas_call(
        _phaseAB_body,
        grid=(NQB, NKB),
        in_specs=[
            pl.BlockSpec((BK, D), lambda ql, kb: (kb, 0)),
            pl.BlockSpec((QL, D), lambda ql, kb: (ql, 0)),
        ],
        out_specs=pl.BlockSpec((1, NCAND, QL), lambda ql, kb: (ql, 0, 0)),
        out_shape=jax.ShapeDtypeStruct((NQB, NCAND, QL), jnp.int32),
        scratch_shapes=[pltpu.VMEM((NCHUNK_PAD, QL), jnp.float32)],
        compiler_params=pltpu.CompilerParams(
            dimension_semantics=("arbitrary", "arbitrary"),
            vmem_limit_bytes=128 * 1024 * 1024,
        ),
    )(kn, qn)


def _pb2_body(q_ref, kc_ref, cand_ref, sel_ref):
    p = lax.dot_general(q_ref[...], kc_ref[...], (((1,), (1,)), ((), ())),
                        preferred_element_type=jnp.float32)
    qrow = lax.broadcasted_iota(jnp.int32, (GQ, NCAND), 0)
    s = jnp.zeros((GQ, NCAND), jnp.float32)
    for a in range(GQ):
        stripe = p[:, a * NCAND:(a + 1) * NCAND]
        s = s + jnp.where(qrow == a, stripe, 0.0)
    cand = cand_ref[...]
    for t in range(K):
        mx = jnp.max(s, axis=1)
        kmask = jnp.where(s == mx[:, None], cand, _IMAX)
        pick = jnp.min(kmask, axis=1)
        sel_ref[:, t] = pick
        if t < K - 1:
            s = jnp.where(kmask == pick[:, None], _NEG, s)


def _pb2(qn, kc, cand):
    return pl.pallas_call(
        _pb2_body,
        grid=(Q // GQ,),
        in_specs=[
            pl.BlockSpec((GQ, D), lambda g: (g, 0)),
            pl.BlockSpec((GQ * NCAND, D), lambda g: (g, 0)),
            pl.BlockSpec((GQ, NCAND), lambda g: (g, 0)),
        ],
        out_specs=pl.BlockSpec((GQ, K), lambda g: (g, 0)),
        out_shape=jax.ShapeDtypeStruct((Q, K), jnp.int32),
    )(qn, kc, cand)


_NW = 32
_QPW = Q // _NW
_SB1 = 8


def _sc_gather_cand(cand_flat, kn_p):
    mesh = plsc.VectorSubcoreMesh(core_axis_name="c", subcore_axis_name="s")

    n = _SB1 * NCAND
    nb = _QPW // _SB1

    @functools.partial(
        pl.kernel, mesh=mesh,
        out_type=jax.ShapeDtypeStruct((Q * NCAND, D), jnp.float32),
        scratch_types=[
            pltpu.VMEM((n,), jnp.int32),
            pltpu.VMEM((n,), jnp.int32),
            pltpu.VMEM((n, D), jnp.float32),
            pltpu.VMEM((n, D), jnp.float32),
            pltpu.SemaphoreType.DMA,
            pltpu.SemaphoreType.DMA,
            pltpu.SemaphoreType.DMA,
            pltpu.SemaphoreType.DMA,
        ],
    )
    def sc1(cand_hbm, kn_hbm, kc_hbm, i0, i1, r0, r1, g0, g1, w0, w1):
        wid = lax.axis_index("s") * 2 + lax.axis_index("c")
        idx, rows, gsem, wsem = (i0, i1), (r0, r1), (g0, g1), (w0, w1)
        gd, wd = {}, {}
        for b in range(nb):
            u = b & 1
            if b >= 2:
                wd[u].wait()
            base = (wid * _QPW + b * _SB1) * NCAND
            pltpu.sync_copy(cand_hbm.at[pl.ds(base, n)], idx[u])
            gd[u] = pltpu.async_copy(kn_hbm.at[idx[u]], rows[u], gsem[u])
            if b >= 1:
                pu = (b - 1) & 1
                gd[pu].wait()
                pbase = (wid * _QPW + (b - 1) * _SB1) * NCAND
                wd[pu] = pltpu.async_copy(rows[pu], kc_hbm.at[pl.ds(pbase, n)],
                                          wsem[pu])
        lu = (nb - 1) & 1
        gd[lu].wait()
        lbase = (wid * _QPW + (nb - 1) * _SB1) * NCAND
        wd[lu] = pltpu.async_copy(rows[lu], kc_hbm.at[pl.ds(lbase, n)], wsem[lu])
        wd[0].wait()
        wd[1].wait()

    return sc1(cand_flat, kn_p)


def _sc_fuse_values(sel_flat, values):
    mesh = plsc.VectorSubcoreMesh(core_axis_name="c", subcore_axis_name="s")
    rpw = _QPW * K

    @functools.partial(
        pl.kernel, mesh=mesh,
        out_type=jax.ShapeDtypeStruct((Q, D), jnp.float32),
        scratch_types=[
            pltpu.VMEM((rpw,), jnp.int32),
            pltpu.VMEM((rpw, D), jnp.float32),
            pltpu.VMEM((_QPW, D), jnp.float32),
            pltpu.SemaphoreType.DMA,
        ],
    )
    def sc2(sel_hbm, val_hbm, out_hbm, sidx_v, vr_v, out_v, sem):
        wid = lax.axis_index("s") * 2 + lax.axis_index("c")
        pltpu.sync_copy(sel_hbm.at[pl.ds(wid * rpw, rpw)], sidx_v)
        pltpu.async_copy(val_hbm.at[sidx_v], vr_v, sem).wait()

        def body(qq, _):
            for d in range(D // 16):
                sl = pl.ds(d * 16, 16)
                acc = (vr_v[qq * K + 0, sl] + vr_v[qq * K + 1, sl]
                       + vr_v[qq * K + 2, sl] + vr_v[qq * K + 3, sl]
                       + vr_v[qq * K + 4, sl])
                out_v[qq, sl] = acc * (1.0 / K)
            return 0

        lax.fori_loop(0, _QPW, body, 0)
        pltpu.sync_copy(out_v, out_hbm.at[pl.ds(wid * _QPW, _QPW)])

    return sc2(sel_flat, values)


def kernel(q, keys, values):
    eps = 1e-8
    qn = q / jnp.maximum(jnp.linalg.norm(q, axis=-1, keepdims=True), eps)
    kn = keys / jnp.maximum(jnp.linalg.norm(keys, axis=-1, keepdims=True), eps)
    kn_p = jnp.pad(kn, ((0, KPAD - CAP), (0, 0)))

    t5 = _phaseAB(qn, kn_p)
    cand = t5.transpose(0, 2, 1).reshape(Q, NCAND)

    kc = _sc_gather_cand(cand.reshape(-1), kn_p)
    sel = _pb2(qn, kc, cand)

    return _sc_fuse_values(sel.reshape(-1), values)


# speedup vs baseline: 6.1255x; 1.0507x over previous
import functools

import jax
import jax.numpy as jnp
from jax import lax
from jax.experimental import pallas as pl
from jax.experimental.pallas import tpu as pltpu
from jax.experimental.pallas import tpu_sc as plsc

CAP = 100000
D = 128
Q = 4096
K = 5
CHUNK = 8
NCAND = K * CHUNK
KPAD = 102400
BK = 2048
NKB = KPAD // BK
CPB = BK // CHUNK
NCHUNK = CAP // CHUNK
NCHUNK_PAD = KPAD // CHUNK
QL = 512
NQB = Q // QL
GQ = 64

_NEG = float("-inf")
_IMAX = 2**31 - 1


def _phaseAB_body(kn_ref, q_ref, out_ref, cmax_ref):
    kb = pl.program_id(1)
    u = lax.dot_general(kn_ref[...], q_ref[...], (((1,), (1,)), ((), ())),
                        preferred_element_type=jnp.float32)
    m = u[0 * CPB:1 * CPB, :]
    for j in range(1, CHUNK):
        m = jnp.maximum(m, u[j * CPB:(j + 1) * CPB, :])
    gchunk = kb * CPB + lax.broadcasted_iota(jnp.int32, (CPB, QL), 0)
    m = jnp.where(gchunk < NCHUNK, m, _NEG)
    cmax_ref[pl.ds(kb * CPB, CPB), :] = m

    @pl.when(kb == NKB - 1)
    def _phaseB():
        rows = lax.broadcasted_iota(jnp.int32, (NCHUNK_PAD, QL), 0)
        for t in range(K):
            c = cmax_ref[...]
            bidx = jnp.argmax(c, axis=0).astype(jnp.int32)
            prow = lax.shift_right_logical(bidx, 8) * BK + (bidx & (CPB - 1))
            for j in range(CHUNK):
                out_ref[0, t * CHUNK + j, :] = prow + j * CPB
            if t < K - 1:
                cmax_ref[...] = jnp.where(rows == bidx[None, :], _NEG, c)


def _phaseAB(qn, kn):
    return pl.pallas_call(
        _phaseAB_body,
        grid=(NQB, NKB),
        in_specs=[
            pl.BlockSpec((BK, D), lambda ql, kb: (kb, 0)),
            pl.BlockSpec((QL, D), lambda ql, kb: (ql, 0)),
        ],
        out_specs=pl.BlockSpec((1, NCAND, QL), lambda ql, kb: (ql, 0, 0)),
        out_shape=jax.ShapeDtypeStruct((NQB, NCAND, QL), jnp.int32),
        scratch_shapes=[pltpu.VMEM((NCHUNK_PAD, QL), jnp.float32)],
        compiler_params=pltpu.CompilerParams(
            dimension_semantics=("arbitrary", "arbitrary"),
            vmem_limit_bytes=128 * 1024 * 1024,
        ),
    )(kn, qn)


def _pb2_body(q_ref, kc_ref, cand_ref, sel_ref):
    p = lax.dot_general(q_ref[...], kc_ref[...], (((1,), (1,)), ((), ())),
                        preferred_element_type=jnp.float32)
    qrow = lax.broadcasted_iota(jnp.int32, (GQ, NCAND), 0)
    s = jnp.zeros((GQ, NCAND), jnp.float32)
    for a in range(GQ):
        stripe = p[:, a * NCAND:(a + 1) * NCAND]
        s = s + jnp.where(qrow == a, stripe, 0.0)
    pid = cand_ref[...]
    cand = ((lax.shift_right_logical(pid, 11) * BK)
            | lax.shift_left((pid & (CPB - 1)), 3)
            | (lax.shift_right_logical(pid, 8) & (CHUNK - 1)))
    for t in range(K):
        mx = jnp.max(s, axis=1)
        kmask = jnp.where(s == mx[:, None], cand, _IMAX)
        pick = jnp.min(kmask, axis=1)
        sel_ref[:, t] = pick
        if t < K - 1:
            s = jnp.where(kmask == pick[:, None], _NEG, s)


def _pb2(qn, kc, cand):
    return pl.pallas_call(
        _pb2_body,
        grid=(Q // GQ,),
        in_specs=[
            pl.BlockSpec((GQ, D), lambda g: (g, 0)),
            pl.BlockSpec((GQ * NCAND, D), lambda g: (g, 0)),
            pl.BlockSpec((GQ, NCAND), lambda g: (g, 0)),
        ],
        out_specs=pl.BlockSpec((GQ, K), lambda g: (g, 0)),
        out_shape=jax.ShapeDtypeStruct((Q, K), jnp.int32),
    )(qn, kc, cand)


_NW = 32
_QPW = Q // _NW
_SB1 = 8


def _sc_gather_cand(cand_flat, kn_p):
    mesh = plsc.VectorSubcoreMesh(core_axis_name="c", subcore_axis_name="s")

    n = _SB1 * NCAND
    nb = _QPW // _SB1

    @functools.partial(
        pl.kernel, mesh=mesh,
        out_type=jax.ShapeDtypeStruct((Q * NCAND, D), jnp.float32),
        scratch_types=[
            pltpu.VMEM((n,), jnp.int32),
            pltpu.VMEM((n,), jnp.int32),
            pltpu.VMEM((n, D), jnp.float32),
            pltpu.VMEM((n, D), jnp.float32),
            pltpu.SemaphoreType.DMA,
            pltpu.SemaphoreType.DMA,
            pltpu.SemaphoreType.DMA,
            pltpu.SemaphoreType.DMA,
        ],
    )
    def sc1(cand_hbm, kn_hbm, kc_hbm, i0, i1, r0, r1, g0, g1, w0, w1):
        wid = lax.axis_index("s") * 2 + lax.axis_index("c")
        idx, rows, gsem, wsem = (i0, i1), (r0, r1), (g0, g1), (w0, w1)
        gd, wd = {}, {}
        for b in range(nb):
            u = b & 1
            if b >= 2:
                wd[u].wait()
            base = (wid * _QPW + b * _SB1) * NCAND
            pltpu.sync_copy(cand_hbm.at[pl.ds(base, n)], idx[u])
            gd[u] = pltpu.async_copy(kn_hbm.at[idx[u]], rows[u], gsem[u])
            if b >= 1:
                pu = (b - 1) & 1
                gd[pu].wait()
                pbase = (wid * _QPW + (b - 1) * _SB1) * NCAND
                wd[pu] = pltpu.async_copy(rows[pu], kc_hbm.at[pl.ds(pbase, n)],
                                          wsem[pu])
        lu = (nb - 1) & 1
        gd[lu].wait()
        lbase = (wid * _QPW + (nb - 1) * _SB1) * NCAND
        wd[lu] = pltpu.async_copy(rows[lu], kc_hbm.at[pl.ds(lbase, n)], wsem[lu])
        wd[0].wait()
        wd[1].wait()

    return sc1(cand_flat, kn_p)


def _sc_fuse_values(sel_flat, values):
    mesh = plsc.VectorSubcoreMesh(core_axis_name="c", subcore_axis_name="s")
    rpw = _QPW * K

    @functools.partial(
        pl.kernel, mesh=mesh,
        out_type=jax.ShapeDtypeStruct((Q, D), jnp.float32),
        scratch_types=[
            pltpu.VMEM((rpw,), jnp.int32),
            pltpu.VMEM((rpw, D), jnp.float32),
            pltpu.VMEM((_QPW, D), jnp.float32),
            pltpu.SemaphoreType.DMA,
        ],
    )
    def sc2(sel_hbm, val_hbm, out_hbm, sidx_v, vr_v, out_v, sem):
        wid = lax.axis_index("s") * 2 + lax.axis_index("c")
        pltpu.sync_copy(sel_hbm.at[pl.ds(wid * rpw, rpw)], sidx_v)
        pltpu.async_copy(val_hbm.at[sidx_v], vr_v, sem).wait()

        def body(qq, _):
            for d in range(D // 16):
                sl = pl.ds(d * 16, 16)
                acc = (vr_v[qq * K + 0, sl] + vr_v[qq * K + 1, sl]
                       + vr_v[qq * K + 2, sl] + vr_v[qq * K + 3, sl]
                       + vr_v[qq * K + 4, sl])
                out_v[qq, sl] = acc * (1.0 / K)
            return 0

        lax.fori_loop(0, _QPW, body, 0)
        pltpu.sync_copy(out_v, out_hbm.at[pl.ds(wid * _QPW, _QPW)])

    return sc2(sel_flat, values)


def kernel(q, keys, values):
    eps = 1e-8
    qn = q / jnp.maximum(jnp.linalg.norm(q, axis=-1, keepdims=True), eps)
    kn = keys / jnp.maximum(jnp.linalg.norm(keys, axis=-1, keepdims=True), eps)
    kn_p = jnp.pad(kn, ((0, KPAD - CAP), (0, 0)))
    kn_perm = (kn_p.reshape(NKB, CPB, CHUNK, D)
               .transpose(0, 2, 1, 3).reshape(KPAD, D))

    t5 = _phaseAB(qn, kn_perm)
    cand = t5.transpose(0, 2, 1).reshape(Q, NCAND)

    kc = _sc_gather_cand(cand.reshape(-1), kn_perm)
    sel = _pb2(qn, kc, cand)

    return _sc_fuse_values(sel.reshape(-1), values)
